# Initial kernel scaffold; baseline (speedup 1.0000x reference)
import jax
import jax.numpy as jnp
from jax.experimental import pallas as pl


def kernel(x, neighs, W1, b1, W2, b2):
    raise NotImplementedError("write your pallas kernel here")


# speedup vs baseline: 1.1993x; 1.1993x over previous
import functools

import jax
import jax.numpy as jnp
from jax import lax
from jax.experimental import pallas as pl
from jax.experimental.pallas import tpu as pltpu
from jax.experimental.pallas import tpu_sc as plsc

N = 10000
DEG = 16
D = 256
LANES = 16
NC = 2
NS = 16
NW = NC * NS
CHUNK = 8
NODES_PER_W = 320
NP = NODES_PER_W * NW
NCHUNKS = NODES_PER_W // CHUNK

_mesh = plsc.VectorSubcoreMesh(core_axis_name="c", subcore_axis_name="s")


def _gmean_body(neighs_hbm, x_hbm, out_hbm, idx_v, rows_v, acc_v, sem):
    wid = lax.axis_index("s") * NC + lax.axis_index("c")
    base = wid * NODES_PER_W

    def chunk_body(k, carry):
        node0 = base + k * CHUNK
        pltpu.sync_copy(neighs_hbm.at[pl.ds(node0 * DEG, CHUNK * DEG)], idx_v)
        pltpu.async_copy(x_hbm.at[idx_v], rows_v, sem).wait()

        def col_body(v, carry2):
            c0 = v * LANES
            for c in range(CHUNK):
                acc = rows_v[c * DEG, pl.ds(c0, LANES)]
                for j in range(1, DEG):
                    acc = acc + rows_v[c * DEG + j, pl.ds(c0, LANES)]
                acc_v[c, pl.ds(c0, LANES)] = acc * (1.0 / DEG)
            return carry2

        lax.fori_loop(0, D // LANES, col_body, 0, unroll=False)
        pltpu.sync_copy(acc_v, out_hbm.at[pl.ds(node0, CHUNK)])
        return carry

    lax.fori_loop(0, NCHUNKS, chunk_body, 0, unroll=False)


@functools.partial(
    pl.kernel,
    out_type=jax.ShapeDtypeStruct((NP, D), jnp.float32),
    mesh=_mesh,
    scratch_types=[
        pltpu.VMEM((CHUNK * DEG,), jnp.int32),
        pltpu.VMEM((CHUNK * DEG, D), jnp.float32),
        pltpu.VMEM((CHUNK, D), jnp.float32),
        pltpu.SemaphoreType.DMA,
    ],
)
def _gmean_sc(neighs_hbm, x_hbm, out_hbm, idx_v, rows_v, acc_v, sem):
    _gmean_body(neighs_hbm, x_hbm, out_hbm, idx_v, rows_v, acc_v, sem)


def _gather_mean(x, neighs_flat):
    return _gmean_sc(neighs_flat, x)


BN = 1000


def _linear_body(h_ref, w_ref, b_ref, o_ref):
    o_ref[...] = (
        lax.dot_general(
            h_ref[...], w_ref[...], (((1,), (1,)), ((), ())),
            preferred_element_type=jnp.float32,
        )
        + b_ref[...]
    )


def _linear(h, W, b):
    return pl.pallas_call(
        _linear_body,
        grid=(N // BN,),
        in_specs=[
            pl.BlockSpec((BN, D), lambda i: (i, 0)),
            pl.BlockSpec((D, D), lambda i: (0, 0)),
            pl.BlockSpec((1, D), lambda i: (0, 0)),
        ],
        out_specs=pl.BlockSpec((BN, D), lambda i: (i, 0)),
        out_shape=jax.ShapeDtypeStruct((N, D), jnp.float32),
    )(h, W, b[None, :])


@jax.jit
def kernel(x, neighs, W1, b1, W2, b2):
    neighs_flat = jnp.pad(neighs, ((0, NP - N), (0, 0))).reshape(-1)
    h1 = _gather_mean(x, neighs_flat)[:N]
    x1 = _linear(h1, W1, b1)
    h2 = _gather_mean(x1, neighs_flat)[:N]
    x2 = _linear(h2, W2, b2)
    return (x1, x2)


# speedup vs baseline: 1.5174x; 1.2652x over previous
import functools

import jax
import jax.numpy as jnp
from jax import lax
from jax.experimental import pallas as pl
from jax.experimental.pallas import tpu as pltpu
from jax.experimental.pallas import tpu_sc as plsc

N = 10000
DEG = 16
D = 256
LANES = 16
NC = 2
NS = 16
NW = NC * NS
CHUNK = 8
NODES_PER_W = 320
NP = NODES_PER_W * NW
NCHUNKS = NODES_PER_W // CHUNK

_mesh = plsc.VectorSubcoreMesh(core_axis_name="c", subcore_axis_name="s")


def _gmean_body(neighs_hbm, x_hbm, out_hbm, idx_all, rows0, rows1,
                acc0, acc1, gsem0, gsem1, osem0, osem1):
    wid = lax.axis_index("s") * NC + lax.axis_index("c")
    base = wid * NODES_PER_W
    rows = (rows0, rows1)
    acc = (acc0, acc1)
    gsem = (gsem0, gsem1)
    osem = (osem0, osem1)

    pltpu.sync_copy(neighs_hbm.at[pl.ds(base * DEG, NODES_PER_W * DEG)],
                    idx_all)

    def start_gather(k, b):
        pltpu.async_copy(
            x_hbm.at[idx_all.at[pl.ds(k * CHUNK * DEG, CHUNK * DEG)]],
            rows[b], gsem[b])

    start_gather(0, 0)

    def outer(k0, carry):
        for b in range(2):
            k = k0 + b

            @pl.when(k + 1 < NCHUNKS)
            def _():
                start_gather(k + 1, 1 - b)

            pltpu.make_async_copy(
                x_hbm.at[idx_all.at[pl.ds(0, CHUNK * DEG)]], rows[b],
                gsem[b]).wait()

            @pl.when(k >= 2)
            def _():
                pltpu.make_async_copy(
                    acc[b], out_hbm.at[pl.ds(base, CHUNK)], osem[b]).wait()

            def col_body(v, carry2):
                c0 = v * LANES
                for c in range(CHUNK):
                    a = rows[b][c * DEG, pl.ds(c0, LANES)]
                    for j in range(1, DEG):
                        a = a + rows[b][c * DEG + j, pl.ds(c0, LANES)]
                    acc[b][c, pl.ds(c0, LANES)] = a * (1.0 / DEG)
                return carry2

            lax.fori_loop(0, D // LANES, col_body, 0, unroll=False)
            pltpu.async_copy(
                acc[b], out_hbm.at[pl.ds(base + k * CHUNK, CHUNK)], osem[b])
        return carry

    lax.fori_loop(0, NCHUNKS // 2, lambda i, c: outer(2 * i, c), 0,
                  unroll=False)
    pltpu.make_async_copy(acc0, out_hbm.at[pl.ds(base, CHUNK)], osem0).wait()
    pltpu.make_async_copy(acc1, out_hbm.at[pl.ds(base, CHUNK)], osem1).wait()


@functools.partial(
    pl.kernel,
    out_type=jax.ShapeDtypeStruct((NP, D), jnp.float32),
    mesh=_mesh,
    scratch_types=[
        pltpu.VMEM((NODES_PER_W * DEG,), jnp.int32),
        pltpu.VMEM((CHUNK * DEG, D), jnp.float32),
        pltpu.VMEM((CHUNK * DEG, D), jnp.float32),
        pltpu.VMEM((CHUNK, D), jnp.float32),
        pltpu.VMEM((CHUNK, D), jnp.float32),
        pltpu.SemaphoreType.DMA,
        pltpu.SemaphoreType.DMA,
        pltpu.SemaphoreType.DMA,
        pltpu.SemaphoreType.DMA,
    ],
)
def _gmean_sc(neighs_hbm, x_hbm, out_hbm, idx_all, rows0, rows1,
              acc0, acc1, gsem0, gsem1, osem0, osem1):
    _gmean_body(neighs_hbm, x_hbm, out_hbm, idx_all, rows0, rows1,
                acc0, acc1, gsem0, gsem1, osem0, osem1)


def _gather_mean(x, neighs_flat):
    return _gmean_sc(neighs_flat, x)


BN = 1000


def _linear_body(h_ref, w_ref, b_ref, o_ref):
    o_ref[...] = (
        lax.dot_general(
            h_ref[...], w_ref[...], (((1,), (1,)), ((), ())),
            preferred_element_type=jnp.float32,
        )
        + b_ref[...]
    )


def _linear(h, W, b):
    return pl.pallas_call(
        _linear_body,
        grid=(N // BN,),
        in_specs=[
            pl.BlockSpec((BN, D), lambda i: (i, 0)),
            pl.BlockSpec((D, D), lambda i: (0, 0)),
            pl.BlockSpec((1, D), lambda i: (0, 0)),
        ],
        out_specs=pl.BlockSpec((BN, D), lambda i: (i, 0)),
        out_shape=jax.ShapeDtypeStruct((N, D), jnp.float32),
    )(h, W, b[None, :])


@jax.jit
def kernel(x, neighs, W1, b1, W2, b2):
    neighs_flat = jnp.pad(neighs, ((0, NP - N), (0, 0))).reshape(-1)
    h1 = _gather_mean(x, neighs_flat)[:N]
    x1 = _linear(h1, W1, b1)
    h2 = _gather_mean(x1, neighs_flat)[:N]
    x2 = _linear(h2, W2, b2)
    return (x1, x2)


# speedup vs baseline: 1.5703x; 1.0349x over previous
import functools

import jax
import jax.numpy as jnp
from jax import lax
from jax.experimental import pallas as pl
from jax.experimental.pallas import tpu as pltpu
from jax.experimental.pallas import tpu_sc as plsc

N = 10000
DEG = 16
D = 256
LANES = 16
SEGS = D // LANES
NC = 2
NS = 16
CHUNK = 8
K0 = 464
K1 = 176
NP = NS * (K0 + K1)
_KMAX = max(K0, K1)

_mesh = plsc.VectorSubcoreMesh(core_axis_name="c", subcore_axis_name="s")


def _gmean_range(base, knodes, neighs_hbm, x_hbm, out_hbm, idx_all,
                 rows, acc, gsem, osem):
    nchunks = knodes // CHUNK

    pltpu.sync_copy(neighs_hbm.at[pl.ds(base * DEG, knodes * DEG)],
                    idx_all.at[pl.ds(0, knodes * DEG)])

    def start_gather(k, b):
        pltpu.async_copy(
            x_hbm.at[idx_all.at[pl.ds(k * CHUNK * DEG, CHUNK * DEG)]],
            rows[b], gsem[b])

    start_gather(0, 0)

    def outer(k0, carry):
        for b in range(2):
            k = k0 + b

            @pl.when(k + 1 < nchunks)
            def _():
                start_gather(k + 1, 1 - b)

            pltpu.make_async_copy(
                x_hbm.at[idx_all.at[pl.ds(0, CHUNK * DEG)]], rows[b],
                gsem[b]).wait()

            @pl.when(k >= 2)
            def _():
                pltpu.make_async_copy(
                    acc[b], out_hbm.at[pl.ds(base, CHUNK)], osem[b]).wait()

            def col_body(v, carry2):
                c0 = v * LANES
                for c in range(CHUNK):
                    a = rows[b][c * DEG, pl.ds(c0, LANES)]
                    for j in range(1, DEG):
                        a = a + rows[b][c * DEG + j, pl.ds(c0, LANES)]
                    acc[b][c, pl.ds(c0, LANES)] = a * (1.0 / DEG)
                return carry2

            lax.fori_loop(0, SEGS, col_body, 0, unroll=False)
            pltpu.async_copy(
                acc[b], out_hbm.at[pl.ds(base + k * CHUNK, CHUNK)], osem[b])
        return carry

    lax.fori_loop(0, nchunks // 2, lambda i, c: outer(2 * i, c), 0,
                  unroll=False)
    pltpu.make_async_copy(acc[0], out_hbm.at[pl.ds(base, CHUNK)],
                          osem[0]).wait()
    pltpu.make_async_copy(acc[1], out_hbm.at[pl.ds(base, CHUNK)],
                          osem[1]).wait()


@functools.partial(
    pl.kernel,
    out_type=jax.ShapeDtypeStruct((NP, D), jnp.float32),
    mesh=_mesh,
    scratch_types=[
        pltpu.VMEM((_KMAX * DEG,), jnp.int32),
        pltpu.VMEM((CHUNK * DEG, D), jnp.float32),
        pltpu.VMEM((CHUNK * DEG, D), jnp.float32),
        pltpu.VMEM((CHUNK, D), jnp.float32),
        pltpu.VMEM((CHUNK, D), jnp.float32),
        pltpu.SemaphoreType.DMA,
        pltpu.SemaphoreType.DMA,
        pltpu.SemaphoreType.DMA,
        pltpu.SemaphoreType.DMA,
    ],
)
def _gmean_sc(neighs_hbm, x_hbm, out_hbm, idx_all, rows0, rows1,
              acc0, acc1, gsem0, gsem1, osem0, osem1):
    c = lax.axis_index("c")
    s = lax.axis_index("s")
    rows = (rows0, rows1)
    acc = (acc0, acc1)
    gsem = (gsem0, gsem1)
    osem = (osem0, osem1)

    @pl.when(c == 0)
    def _():
        _gmean_range(s * K0, K0, neighs_hbm, x_hbm, out_hbm, idx_all,
                     rows, acc, gsem, osem)

    @pl.when(c == 1)
    def _():
        _gmean_range(NS * K0 + s * K1, K1, neighs_hbm, x_hbm, out_hbm,
                     idx_all, rows, acc, gsem, osem)


BN = 1000


def _linear_body(h_ref, w_ref, b_ref, o_ref):
    o_ref[...] = (
        lax.dot_general(
            h_ref[...], w_ref[...], (((1,), (1,)), ((), ())),
            preferred_element_type=jnp.float32,
        )
        + b_ref[...]
    )


def _linear(h, W, b):
    return pl.pallas_call(
        _linear_body,
        grid=(N // BN,),
        in_specs=[
            pl.BlockSpec((BN, D), lambda i: (i, 0)),
            pl.BlockSpec((D, D), lambda i: (0, 0)),
            pl.BlockSpec((1, D), lambda i: (0, 0)),
        ],
        out_specs=pl.BlockSpec((BN, D), lambda i: (i, 0)),
        out_shape=jax.ShapeDtypeStruct((N, D), jnp.float32),
    )(h, W, b[None, :])


@jax.jit
def kernel(x, neighs, W1, b1, W2, b2):
    neighs_flat = jnp.pad(neighs, ((0, NP - N), (0, 0))).reshape(-1)
    h1 = _gmean_sc(neighs_flat, x)[:N]
    x1 = _linear(h1, W1, b1)
    h2 = _gmean_sc(neighs_flat, x1)[:N]
    x2 = _linear(h2, W2, b2)
    return (x1, x2)


# speedup vs baseline: 3.1500x; 2.0059x over previous
import functools

import jax
import jax.numpy as jnp
from jax import lax
from jax.experimental import pallas as pl
from jax.experimental.pallas import tpu as pltpu
from jax.experimental.pallas import tpu_sc as plsc

N = 10000
DEG = 16
D = 256
HALF = D // 2
LANES = 16
SEGS = HALF // LANES
NC = 2
NS = 16
CHUNK = 8
NPT = 640
NP = NS * NPT
NCHUNKS = NPT // CHUNK
RPT = 632
RPT_LAST = N - (NS - 1) * RPT

_mesh = plsc.VectorSubcoreMesh(core_axis_name="c", subcore_axis_name="s")


def _gmean_half(cbase, s, neighs_hbm, x_hbm, out_hbm, idx_all, shared,
                rows, acc, gsem, osem):
    @pl.when(s < NS - 1)
    def _():
        pltpu.sync_copy(
            x_hbm.at[pl.ds(s * RPT, RPT), pl.ds(cbase, HALF)],
            shared.at[pl.ds(s * RPT, RPT)])

    @pl.when(s == NS - 1)
    def _():
        pltpu.sync_copy(
            x_hbm.at[pl.ds((NS - 1) * RPT, RPT_LAST), pl.ds(cbase, HALF)],
            shared.at[pl.ds((NS - 1) * RPT, RPT_LAST)])

    plsc.subcore_barrier()

    base = s * NPT
    pltpu.sync_copy(neighs_hbm.at[pl.ds(base * DEG, NPT * DEG)], idx_all)

    def start_gather(k, b):
        pltpu.async_copy(
            shared.at[idx_all.at[pl.ds(k * CHUNK * DEG, CHUNK * DEG)]],
            rows[b], gsem[b])

    start_gather(0, 0)

    def outer(k0, carry):
        for b in range(2):
            k = k0 + b

            @pl.when(k + 1 < NCHUNKS)
            def _():
                start_gather(k + 1, 1 - b)

            pltpu.make_async_copy(
                shared.at[idx_all.at[pl.ds(0, CHUNK * DEG)]], rows[b],
                gsem[b]).wait()

            @pl.when(k >= 2)
            def _():
                pltpu.make_async_copy(
                    acc[b],
                    out_hbm.at[pl.ds(base, CHUNK), pl.ds(cbase, HALF)],
                    osem[b]).wait()

            def col_body(v, carry2):
                c0 = v * LANES
                for c in range(CHUNK):
                    a = rows[b][c * DEG, pl.ds(c0, LANES)]
                    for j in range(1, DEG):
                        a = a + rows[b][c * DEG + j, pl.ds(c0, LANES)]
                    acc[b][c, pl.ds(c0, LANES)] = a * (1.0 / DEG)
                return carry2

            lax.fori_loop(0, SEGS, col_body, 0, unroll=False)
            pltpu.async_copy(
                acc[b],
                out_hbm.at[pl.ds(base + k * CHUNK, CHUNK),
                           pl.ds(cbase, HALF)],
                osem[b])
        return carry

    lax.fori_loop(0, NCHUNKS // 2, lambda i, c: outer(2 * i, c), 0,
                  unroll=False)
    pltpu.make_async_copy(
        acc[0], out_hbm.at[pl.ds(base, CHUNK), pl.ds(cbase, HALF)],
        osem[0]).wait()
    pltpu.make_async_copy(
        acc[1], out_hbm.at[pl.ds(base, CHUNK), pl.ds(cbase, HALF)],
        osem[1]).wait()


@functools.partial(
    pl.kernel,
    out_type=jax.ShapeDtypeStruct((NP, D), jnp.float32),
    mesh=_mesh,
    scratch_types=[
        pltpu.VMEM((NPT * DEG,), jnp.int32),
        pltpu.VMEM_SHARED((N, HALF), jnp.float32),
        pltpu.VMEM((CHUNK * DEG, HALF), jnp.float32),
        pltpu.VMEM((CHUNK * DEG, HALF), jnp.float32),
        pltpu.VMEM((CHUNK, HALF), jnp.float32),
        pltpu.VMEM((CHUNK, HALF), jnp.float32),
        pltpu.SemaphoreType.DMA,
        pltpu.SemaphoreType.DMA,
        pltpu.SemaphoreType.DMA,
        pltpu.SemaphoreType.DMA,
    ],
)
def _gmean_sc(neighs_hbm, x_hbm, out_hbm, idx_all, shared, rows0, rows1,
              acc0, acc1, gsem0, gsem1, osem0, osem1):
    c = lax.axis_index("c")
    s = lax.axis_index("s")
    rows = (rows0, rows1)
    acc = (acc0, acc1)
    gsem = (gsem0, gsem1)
    osem = (osem0, osem1)

    @pl.when(c == 0)
    def _():
        _gmean_half(0, s, neighs_hbm, x_hbm, out_hbm, idx_all, shared,
                    rows, acc, gsem, osem)

    @pl.when(c == 1)
    def _():
        _gmean_half(HALF, s, neighs_hbm, x_hbm, out_hbm, idx_all, shared,
                    rows, acc, gsem, osem)


BN = 1000


def _linear_body(h_ref, w_ref, b_ref, o_ref):
    o_ref[...] = (
        lax.dot_general(
            h_ref[...], w_ref[...], (((1,), (1,)), ((), ())),
            preferred_element_type=jnp.float32,
        )
        + b_ref[...]
    )


def _linear(h, W, b):
    return pl.pallas_call(
        _linear_body,
        grid=(N // BN,),
        in_specs=[
            pl.BlockSpec((BN, D), lambda i: (i, 0)),
            pl.BlockSpec((D, D), lambda i: (0, 0)),
            pl.BlockSpec((1, D), lambda i: (0, 0)),
        ],
        out_specs=pl.BlockSpec((BN, D), lambda i: (i, 0)),
        out_shape=jax.ShapeDtypeStruct((N, D), jnp.float32),
    )(h, W, b[None, :])


@jax.jit
def kernel(x, neighs, W1, b1, W2, b2):
    neighs_flat = jnp.pad(neighs, ((0, NP - N), (0, 0))).reshape(-1)
    h1 = _gmean_sc(neighs_flat, x)[:N]
    x1 = _linear(h1, W1, b1)
    h2 = _gmean_sc(neighs_flat, x1)[:N]
    x2 = _linear(h2, W2, b2)
    return (x1, x2)


# speedup vs baseline: 3.3346x; 1.0586x over previous
import functools

import jax
import jax.numpy as jnp
from jax import lax
from jax.experimental import pallas as pl
from jax.experimental.pallas import tpu as pltpu
from jax.experimental.pallas import tpu_sc as plsc

N = 10000
DEG = 16
D = 256
HALF = D // 2
LANES = 16
SEGS = HALF // LANES
NC = 2
NS = 16
CHUNK = 8
NPT = 640
NP = NS * NPT
NCHUNKS = NPT // CHUNK
RPT = 632
RPT_LAST = N - (NS - 1) * RPT

_mesh = plsc.VectorSubcoreMesh(core_axis_name="c", subcore_axis_name="s")


def _gmean_half(cbase, s, neighs_hbm, x_hbm, out_hbm, idx_all, shared,
                rows, acc, gsem, osem):
    @pl.when(s < NS - 1)
    def _():
        pltpu.sync_copy(
            x_hbm.at[pl.ds(s * RPT, RPT), pl.ds(cbase, HALF)],
            shared.at[pl.ds(s * RPT, RPT)])

    @pl.when(s == NS - 1)
    def _():
        pltpu.sync_copy(
            x_hbm.at[pl.ds((NS - 1) * RPT, RPT_LAST), pl.ds(cbase, HALF)],
            shared.at[pl.ds((NS - 1) * RPT, RPT_LAST)])

    plsc.subcore_barrier()

    base = s * NPT
    pltpu.sync_copy(neighs_hbm.at[pl.ds(base * DEG, NPT * DEG)], idx_all)

    def start_gather(k, b):
        pltpu.async_copy(
            shared.at[idx_all.at[pl.ds(k * CHUNK * DEG, CHUNK * DEG)]],
            rows[b], gsem[b])

    start_gather(0, 0)

    def outer(k0, carry):
        for b in range(2):
            k = k0 + b

            @pl.when(k + 1 < NCHUNKS)
            def _():
                start_gather(k + 1, 1 - b)

            pltpu.make_async_copy(
                shared.at[idx_all.at[pl.ds(0, CHUNK * DEG)]], rows[b],
                gsem[b]).wait()

            @pl.when(k >= 2)
            def _():
                pltpu.make_async_copy(
                    acc[b],
                    out_hbm.at[pl.ds(base, CHUNK), pl.ds(cbase, HALF)],
                    osem[b]).wait()

            def col_body(v, carry2):
                c0 = v * LANES
                for c in range(CHUNK):
                    vals = [rows[b][c * DEG + j, pl.ds(c0, LANES)]
                            for j in range(DEG)]
                    while len(vals) > 1:
                        vals = [vals[i] + vals[i + 1]
                                for i in range(0, len(vals), 2)]
                    acc[b][c, pl.ds(c0, LANES)] = vals[0] * (1.0 / DEG)
                return carry2

            lax.fori_loop(0, SEGS, col_body, 0, unroll=False)
            pltpu.async_copy(
                acc[b],
                out_hbm.at[pl.ds(base + k * CHUNK, CHUNK),
                           pl.ds(cbase, HALF)],
                osem[b])
        return carry

    lax.fori_loop(0, NCHUNKS // 2, lambda i, c: outer(2 * i, c), 0,
                  unroll=False)
    pltpu.make_async_copy(
        acc[0], out_hbm.at[pl.ds(base, CHUNK), pl.ds(cbase, HALF)],
        osem[0]).wait()
    pltpu.make_async_copy(
        acc[1], out_hbm.at[pl.ds(base, CHUNK), pl.ds(cbase, HALF)],
        osem[1]).wait()


@functools.partial(
    pl.kernel,
    out_type=jax.ShapeDtypeStruct((NP, D), jnp.float32),
    mesh=_mesh,
    scratch_types=[
        pltpu.VMEM((NPT * DEG,), jnp.int32),
        pltpu.VMEM_SHARED((N, HALF), jnp.float32),
        pltpu.VMEM((CHUNK * DEG, HALF), jnp.float32),
        pltpu.VMEM((CHUNK * DEG, HALF), jnp.float32),
        pltpu.VMEM((CHUNK, HALF), jnp.float32),
        pltpu.VMEM((CHUNK, HALF), jnp.float32),
        pltpu.SemaphoreType.DMA,
        pltpu.SemaphoreType.DMA,
        pltpu.SemaphoreType.DMA,
        pltpu.SemaphoreType.DMA,
    ],
)
def _gmean_sc(neighs_hbm, x_hbm, out_hbm, idx_all, shared, rows0, rows1,
              acc0, acc1, gsem0, gsem1, osem0, osem1):
    c = lax.axis_index("c")
    s = lax.axis_index("s")
    rows = (rows0, rows1)
    acc = (acc0, acc1)
    gsem = (gsem0, gsem1)
    osem = (osem0, osem1)

    @pl.when(c == 0)
    def _():
        _gmean_half(0, s, neighs_hbm, x_hbm, out_hbm, idx_all, shared,
                    rows, acc, gsem, osem)

    @pl.when(c == 1)
    def _():
        _gmean_half(HALF, s, neighs_hbm, x_hbm, out_hbm, idx_all, shared,
                    rows, acc, gsem, osem)


BN = 1000


def _linear_body(h_ref, w_ref, b_ref, o_ref):
    o_ref[...] = (
        lax.dot_general(
            h_ref[...], w_ref[...], (((1,), (1,)), ((), ())),
            preferred_element_type=jnp.float32,
        )
        + b_ref[...]
    )


def _linear(h_padded, W, b):
    return pl.pallas_call(
        _linear_body,
        grid=(N // BN,),
        in_specs=[
            pl.BlockSpec((BN, D), lambda i: (i, 0)),
            pl.BlockSpec((D, D), lambda i: (0, 0)),
            pl.BlockSpec((1, D), lambda i: (0, 0)),
        ],
        out_specs=pl.BlockSpec((BN, D), lambda i: (i, 0)),
        out_shape=jax.ShapeDtypeStruct((N, D), jnp.float32),
    )(h_padded, W, b[None, :])


@jax.jit
def kernel(x, neighs, W1, b1, W2, b2):
    neighs_flat = jnp.pad(neighs, ((0, NP - N), (0, 0))).reshape(-1)
    h1 = _gmean_sc(neighs_flat, x)
    x1 = _linear(h1, W1, b1)
    h2 = _gmean_sc(neighs_flat, x1)
    x2 = _linear(h2, W2, b2)
    return (x1, x2)


# speedup vs baseline: 3.3418x; 1.0022x over previous
import functools

import jax
import jax.numpy as jnp
from jax import lax
from jax.experimental import pallas as pl
from jax.experimental.pallas import tpu as pltpu
from jax.experimental.pallas import tpu_sc as plsc

N = 10000
DEG = 16
D = 256
HALF = D // 2
LANES = 16
SEGS = HALF // LANES
NC = 2
NS = 16
CHUNK = 8
NBUF = 2
NPT = 640
NP = NS * NPT
NCHUNKS = NPT // CHUNK
RPT = 632
VALID_LAST = N - (NS - 1) * NPT
RPT_LAST = N - (NS - 1) * RPT

_mesh = plsc.VectorSubcoreMesh(core_axis_name="c", subcore_axis_name="s")


def _gmean_half(cbase, s, neighs_hbm, x_hbm, out_hbm, idx_all, shared,
                rows, acc, gsem, osem):
    @pl.when(s < NS - 1)
    def _():
        pltpu.sync_copy(
            x_hbm.at[pl.ds(s * RPT, RPT), pl.ds(cbase, HALF)],
            shared.at[pl.ds(s * RPT, RPT)])

    @pl.when(s == NS - 1)
    def _():
        pltpu.sync_copy(
            x_hbm.at[pl.ds((NS - 1) * RPT, RPT_LAST), pl.ds(cbase, HALF)],
            shared.at[pl.ds((NS - 1) * RPT, RPT_LAST)])

    plsc.subcore_barrier()

    base = s * NPT
    @pl.when(s < NS - 1)
    def _():
        pltpu.sync_copy(neighs_hbm.at[pl.ds(base * DEG, NPT * DEG)],
                        idx_all)

    @pl.when(s == NS - 1)
    def _():
        pltpu.sync_copy(
            neighs_hbm.at[pl.ds((NS - 1) * NPT * DEG, VALID_LAST * DEG)],
            idx_all.at[pl.ds(0, VALID_LAST * DEG)])
        zeros = jnp.zeros((LANES,), jnp.int32)

        def zero_body(i, cr):
            idx_all[pl.ds(VALID_LAST * DEG + i * LANES, LANES)] = zeros
            return cr

        lax.fori_loop(0, (NPT - VALID_LAST) * DEG // LANES, zero_body, 0,
                      unroll=False)

    def start_gather(k, b):
        pltpu.async_copy(
            shared.at[idx_all.at[pl.ds(k * CHUNK * DEG, CHUNK * DEG)]],
            rows[b], gsem[b])

    for kp in range(NBUF - 1):
        start_gather(kp, kp)

    def outer(k0, carry):
        for b in range(NBUF):
            k = k0 + b

            @pl.when(k + NBUF - 1 < NCHUNKS)
            def _():
                start_gather(k + NBUF - 1, (b + NBUF - 1) % NBUF)

            pltpu.make_async_copy(
                shared.at[idx_all.at[pl.ds(0, CHUNK * DEG)]], rows[b],
                gsem[b]).wait()

            @pl.when(k >= NBUF)
            def _():
                pltpu.make_async_copy(
                    acc[b],
                    out_hbm.at[pl.ds(base, CHUNK), pl.ds(cbase, HALF)],
                    osem[b]).wait()

            def col_body(v, carry2):
                c0 = v * LANES
                for c in range(CHUNK):
                    vals = [rows[b][c * DEG + j, pl.ds(c0, LANES)]
                            for j in range(DEG)]
                    while len(vals) > 1:
                        vals = [vals[i] + vals[i + 1]
                                for i in range(0, len(vals), 2)]
                    acc[b][c, pl.ds(c0, LANES)] = vals[0] * (1.0 / DEG)
                return carry2

            lax.fori_loop(0, SEGS, col_body, 0, unroll=False)
            pltpu.async_copy(
                acc[b],
                out_hbm.at[pl.ds(base + k * CHUNK, CHUNK),
                           pl.ds(cbase, HALF)],
                osem[b])
        return carry

    lax.fori_loop(0, NCHUNKS // NBUF, lambda i, c: outer(NBUF * i, c), 0,
                  unroll=False)
    for b in range(NBUF):
        pltpu.make_async_copy(
            acc[b], out_hbm.at[pl.ds(base, CHUNK), pl.ds(cbase, HALF)],
            osem[b]).wait()


@functools.partial(
    pl.kernel,
    out_type=jax.ShapeDtypeStruct((NP, D), jnp.float32),
    mesh=_mesh,
    scratch_types=[
        pltpu.VMEM((NPT * DEG,), jnp.int32),
        pltpu.VMEM_SHARED((N, HALF), jnp.float32),
        *[pltpu.VMEM((CHUNK * DEG, HALF), jnp.float32)
          for _ in range(NBUF)],
        *[pltpu.VMEM((CHUNK, HALF), jnp.float32) for _ in range(NBUF)],
        *[pltpu.SemaphoreType.DMA for _ in range(2 * NBUF)],
    ],
)
def _gmean_sc(neighs_hbm, x_hbm, out_hbm, idx_all, shared, *bufs):
    rows = tuple(bufs[0:NBUF])
    acc = tuple(bufs[NBUF:2 * NBUF])
    gsem = tuple(bufs[2 * NBUF:3 * NBUF])
    osem = tuple(bufs[3 * NBUF:4 * NBUF])
    c = lax.axis_index("c")
    s = lax.axis_index("s")

    @pl.when(c == 0)
    def _():
        _gmean_half(0, s, neighs_hbm, x_hbm, out_hbm, idx_all, shared,
                    rows, acc, gsem, osem)

    @pl.when(c == 1)
    def _():
        _gmean_half(HALF, s, neighs_hbm, x_hbm, out_hbm, idx_all, shared,
                    rows, acc, gsem, osem)


BN = 1000


def _linear_body(h_ref, w_ref, b_ref, o_ref):
    o_ref[...] = (
        lax.dot_general(
            h_ref[...].astype(jnp.bfloat16),
            w_ref[...].astype(jnp.bfloat16),
            (((1,), (1,)), ((), ())),
            preferred_element_type=jnp.float32,
        )
        + b_ref[...]
    )


def _linear(h_padded, W, b):
    return pl.pallas_call(
        _linear_body,
        grid=(N // BN,),
        in_specs=[
            pl.BlockSpec((BN, D), lambda i: (i, 0)),
            pl.BlockSpec((D, D), lambda i: (0, 0)),
            pl.BlockSpec((1, D), lambda i: (0, 0)),
        ],
        out_specs=pl.BlockSpec((BN, D), lambda i: (i, 0)),
        out_shape=jax.ShapeDtypeStruct((N, D), jnp.float32),
    )(h_padded, W, b[None, :])


@jax.jit
def kernel(x, neighs, W1, b1, W2, b2):
    neighs_flat = neighs.reshape(-1)
    h1 = _gmean_sc(neighs_flat, x)
    x1 = _linear(h1, W1, b1)
    h2 = _gmean_sc(neighs_flat, x1)
    x2 = _linear(h2, W2, b2)
    return (x1, x2)


# speedup vs baseline: 3.6511x; 1.0925x over previous
import functools

import jax
import jax.numpy as jnp
from jax import lax
from jax.experimental import pallas as pl
from jax.experimental.pallas import tpu as pltpu
from jax.experimental.pallas import tpu_sc as plsc

N = 10000
DEG = 16
D = 256
HALF = D // 2
LANES = 16
SEGS = HALF // LANES
NC = 2
NS = 16
CHUNK = 8
NBUF = 2
NSC = 8192
NPT = NSC // NS
NCHUNKS = NPT // CHUNK
NTC = N - NSC
RPT = 632
RPT_LAST = N - (NS - 1) * RPT

_mesh = plsc.VectorSubcoreMesh(core_axis_name="c", subcore_axis_name="s")


def _gmean_half(cbase, s, neighs_hbm, x_hbm, out_hbm, idx_all, shared,
                rows, acc, gsem, osem):
    @pl.when(s < NS - 1)
    def _():
        pltpu.sync_copy(
            x_hbm.at[pl.ds(s * RPT, RPT), pl.ds(cbase, HALF)],
            shared.at[pl.ds(s * RPT, RPT)])

    @pl.when(s == NS - 1)
    def _():
        pltpu.sync_copy(
            x_hbm.at[pl.ds((NS - 1) * RPT, RPT_LAST), pl.ds(cbase, HALF)],
            shared.at[pl.ds((NS - 1) * RPT, RPT_LAST)])

    plsc.subcore_barrier()

    base = s * NPT
    pltpu.sync_copy(neighs_hbm.at[pl.ds(base * DEG, NPT * DEG)], idx_all)

    def start_gather(k, b):
        pltpu.async_copy(
            shared.at[idx_all.at[pl.ds(k * CHUNK * DEG, CHUNK * DEG)]],
            rows[b], gsem[b])

    for kp in range(NBUF - 1):
        start_gather(kp, kp)

    def outer(k0, carry):
        for b in range(NBUF):
            k = k0 + b

            @pl.when(k + NBUF - 1 < NCHUNKS)
            def _():
                start_gather(k + NBUF - 1, (b + NBUF - 1) % NBUF)

            pltpu.make_async_copy(
                shared.at[idx_all.at[pl.ds(0, CHUNK * DEG)]], rows[b],
                gsem[b]).wait()

            @pl.when(k >= NBUF)
            def _():
                pltpu.make_async_copy(
                    acc[b],
                    out_hbm.at[pl.ds(base, CHUNK), pl.ds(cbase, HALF)],
                    osem[b]).wait()

            def col_body(v, carry2):
                c0 = v * LANES
                for c in range(CHUNK):
                    vals = [rows[b][c * DEG + j, pl.ds(c0, LANES)]
                            for j in range(DEG)]
                    while len(vals) > 1:
                        vals = [vals[i] + vals[i + 1]
                                for i in range(0, len(vals), 2)]
                    acc[b][c, pl.ds(c0, LANES)] = vals[0] * (1.0 / DEG)
                return carry2

            lax.fori_loop(0, SEGS, col_body, 0, unroll=False)
            pltpu.async_copy(
                acc[b],
                out_hbm.at[pl.ds(base + k * CHUNK, CHUNK),
                           pl.ds(cbase, HALF)],
                osem[b])
        return carry

    lax.fori_loop(0, NCHUNKS // NBUF, lambda i, c: outer(NBUF * i, c), 0,
                  unroll=False)
    for b in range(NBUF):
        pltpu.make_async_copy(
            acc[b], out_hbm.at[pl.ds(base, CHUNK), pl.ds(cbase, HALF)],
            osem[b]).wait()


@functools.partial(
    pl.kernel,
    out_type=jax.ShapeDtypeStruct((NSC, D), jnp.float32),
    mesh=_mesh,
    scratch_types=[
        pltpu.VMEM((NPT * DEG,), jnp.int32),
        pltpu.VMEM_SHARED((N, HALF), jnp.float32),
        *[pltpu.VMEM((CHUNK * DEG, HALF), jnp.float32)
          for _ in range(NBUF)],
        *[pltpu.VMEM((CHUNK, HALF), jnp.float32) for _ in range(NBUF)],
        *[pltpu.SemaphoreType.DMA for _ in range(2 * NBUF)],
    ],
)
def _gmean_sc(neighs_hbm, x_hbm, out_hbm, idx_all, shared, *bufs):
    rows = tuple(bufs[0:NBUF])
    acc = tuple(bufs[NBUF:2 * NBUF])
    gsem = tuple(bufs[2 * NBUF:3 * NBUF])
    osem = tuple(bufs[3 * NBUF:4 * NBUF])
    c = lax.axis_index("c")
    s = lax.axis_index("s")

    @pl.when(c == 0)
    def _():
        _gmean_half(0, s, neighs_hbm, x_hbm, out_hbm, idx_all, shared,
                    rows, acc, gsem, osem)

    @pl.when(c == 1)
    def _():
        _gmean_half(HALF, s, neighs_hbm, x_hbm, out_hbm, idx_all, shared,
                    rows, acc, gsem, osem)


BN = 1000


def _linear_body(h_ref, w_ref, b_ref, o_ref):
    o_ref[...] = (
        lax.dot_general(
            h_ref[...].astype(jnp.bfloat16),
            w_ref[...].astype(jnp.bfloat16),
            (((1,), (1,)), ((), ())),
            preferred_element_type=jnp.float32,
        )
        + b_ref[...]
    )


def _linear(h, W, b):
    return pl.pallas_call(
        _linear_body,
        grid=(N // BN,),
        in_specs=[
            pl.BlockSpec((BN, D), lambda i: (i, 0)),
            pl.BlockSpec((D, D), lambda i: (0, 0)),
            pl.BlockSpec((1, D), lambda i: (0, 0)),
        ],
        out_specs=pl.BlockSpec((BN, D), lambda i: (i, 0)),
        out_shape=jax.ShapeDtypeStruct((N, D), jnp.float32),
    )(h, W, b[None, :])


TCB = 904


def _gmean_tc_body(idx_ref, x_ref, o_ref):
    def node_body(n, carry):
        vals = [x_ref[pl.ds(idx_ref[0, 0, n * DEG + j], 1), :]
                for j in range(DEG)]
        while len(vals) > 1:
            vals = [vals[i] + vals[i + 1] for i in range(0, len(vals), 2)]
        o_ref[pl.ds(n, 1), :] = vals[0] * (1.0 / DEG)
        return carry

    lax.fori_loop(0, TCB, node_body, 0, unroll=False)


def _gmean_tc(neighs_tail, x):
    return pl.pallas_call(
        _gmean_tc_body,
        grid=(NTC // TCB,),
        in_specs=[
            pl.BlockSpec((1, 1, TCB * DEG), lambda i: (i, 0, 0),
                         memory_space=pltpu.SMEM),
            pl.BlockSpec((N, D), lambda i: (0, 0)),
        ],
        out_specs=pl.BlockSpec((TCB, D), lambda i: (i, 0)),
        out_shape=jax.ShapeDtypeStruct((NTC, D), jnp.float32),
    )(neighs_tail.reshape(NTC // TCB, 1, TCB * DEG), x)


@jax.jit
def kernel(x, neighs, W1, b1, W2, b2):
    neighs_sc = neighs[:NSC].reshape(-1)
    neighs_tc = neighs[NSC:]
    h1_sc = _gmean_sc(neighs_sc, x)
    h1_tc = _gmean_tc(neighs_tc, x)
    h1 = jnp.concatenate([h1_sc, h1_tc], axis=0)
    x1 = _linear(h1, W1, b1)
    h2_sc = _gmean_sc(neighs_sc, x1)
    h2_tc = _gmean_tc(neighs_tc, x1)
    h2 = jnp.concatenate([h2_sc, h2_tc], axis=0)
    x2 = _linear(h2, W2, b2)
    return (x1, x2)


# speedup vs baseline: 3.7451x; 1.0258x over previous
import functools

import jax
import jax.numpy as jnp
from jax import lax
from jax.experimental import pallas as pl
from jax.experimental.pallas import tpu as pltpu
from jax.experimental.pallas import tpu_sc as plsc

N = 10000
DEG = 16
D = 256
HALF = D // 2
LANES = 16
SEGS = HALF // LANES
NC = 2
NS = 16
CHUNK = 8
NBUF = 2
NSC = 7936
NPT = NSC // NS
NCHUNKS = NPT // CHUNK
NTC = N - NSC
RPT = 632
RPT_LAST = N - (NS - 1) * RPT

_mesh = plsc.VectorSubcoreMesh(core_axis_name="c", subcore_axis_name="s")


def _gmean_half(cbase, s, neighs_hbm, x_hbm, out_hbm, idx_all, shared,
                rows, acc, gsem, osem):
    @pl.when(s < NS - 1)
    def _():
        pltpu.sync_copy(
            x_hbm.at[pl.ds(s * RPT, RPT), pl.ds(cbase, HALF)],
            shared.at[pl.ds(s * RPT, RPT)])

    @pl.when(s == NS - 1)
    def _():
        pltpu.sync_copy(
            x_hbm.at[pl.ds((NS - 1) * RPT, RPT_LAST), pl.ds(cbase, HALF)],
            shared.at[pl.ds((NS - 1) * RPT, RPT_LAST)])

    plsc.subcore_barrier()

    base = s * NPT
    pltpu.sync_copy(neighs_hbm.at[pl.ds(base * DEG, NPT * DEG)], idx_all)

    def start_gather(k, b):
        pltpu.async_copy(
            shared.at[idx_all.at[pl.ds(k * CHUNK * DEG, CHUNK * DEG)]],
            rows[b], gsem[b])

    for kp in range(NBUF - 1):
        start_gather(kp, kp)

    def outer(k0, carry):
        for b in range(NBUF):
            k = k0 + b

            @pl.when(k + NBUF - 1 < NCHUNKS)
            def _():
                start_gather(k + NBUF - 1, (b + NBUF - 1) % NBUF)

            pltpu.make_async_copy(
                shared.at[idx_all.at[pl.ds(0, CHUNK * DEG)]], rows[b],
                gsem[b]).wait()

            @pl.when(k >= NBUF)
            def _():
                pltpu.make_async_copy(
                    acc[b],
                    out_hbm.at[pl.ds(base, CHUNK), pl.ds(cbase, HALF)],
                    osem[b]).wait()

            def col_body(v, carry2):
                c0 = v * LANES
                for c in range(CHUNK):
                    vals = [rows[b][c * DEG + j, pl.ds(c0, LANES)]
                            for j in range(DEG)]
                    while len(vals) > 1:
                        vals = [vals[i] + vals[i + 1]
                                for i in range(0, len(vals), 2)]
                    acc[b][c, pl.ds(c0, LANES)] = vals[0] * (1.0 / DEG)
                return carry2

            lax.fori_loop(0, SEGS, col_body, 0, unroll=False)
            pltpu.async_copy(
                acc[b],
                out_hbm.at[pl.ds(base + k * CHUNK, CHUNK),
                           pl.ds(cbase, HALF)],
                osem[b])
        return carry

    lax.fori_loop(0, NCHUNKS // NBUF, lambda i, c: outer(NBUF * i, c), 0,
                  unroll=False)
    for b in range(NBUF):
        pltpu.make_async_copy(
            acc[b], out_hbm.at[pl.ds(base, CHUNK), pl.ds(cbase, HALF)],
            osem[b]).wait()


@functools.partial(
    pl.kernel,
    out_type=jax.ShapeDtypeStruct((NSC, D), jnp.float32),
    mesh=_mesh,
    scratch_types=[
        pltpu.VMEM((NPT * DEG,), jnp.int32),
        pltpu.VMEM_SHARED((N, HALF), jnp.float32),
        *[pltpu.VMEM((CHUNK * DEG, HALF), jnp.float32)
          for _ in range(NBUF)],
        *[pltpu.VMEM((CHUNK, HALF), jnp.float32) for _ in range(NBUF)],
        *[pltpu.SemaphoreType.DMA for _ in range(2 * NBUF)],
    ],
)
def _gmean_sc(neighs_hbm, x_hbm, out_hbm, idx_all, shared, *bufs):
    rows = tuple(bufs[0:NBUF])
    acc = tuple(bufs[NBUF:2 * NBUF])
    gsem = tuple(bufs[2 * NBUF:3 * NBUF])
    osem = tuple(bufs[3 * NBUF:4 * NBUF])
    c = lax.axis_index("c")
    s = lax.axis_index("s")

    @pl.when(c == 0)
    def _():
        _gmean_half(0, s, neighs_hbm, x_hbm, out_hbm, idx_all, shared,
                    rows, acc, gsem, osem)

    @pl.when(c == 1)
    def _():
        _gmean_half(HALF, s, neighs_hbm, x_hbm, out_hbm, idx_all, shared,
                    rows, acc, gsem, osem)


BN = 1000


def _linear_body(h_ref, w_ref, b_ref, o_ref):
    o_ref[...] = (
        lax.dot_general(
            h_ref[...].astype(jnp.bfloat16),
            w_ref[...].astype(jnp.bfloat16),
            (((1,), (1,)), ((), ())),
            preferred_element_type=jnp.float32,
        )
        + b_ref[...]
    )


def _linear(h, W, b):
    return pl.pallas_call(
        _linear_body,
        grid=(N // BN,),
        in_specs=[
            pl.BlockSpec((BN, D), lambda i: (i, 0)),
            pl.BlockSpec((D, D), lambda i: (0, 0)),
            pl.BlockSpec((1, D), lambda i: (0, 0)),
        ],
        out_specs=pl.BlockSpec((BN, D), lambda i: (i, 0)),
        out_shape=jax.ShapeDtypeStruct((N, D), jnp.float32),
    )(h, W, b[None, :])


TCB = 344


def _gmean_tc_body(idx_ref, x_ref, o_ref):
    def node_body(n, carry):
        vals = [x_ref[pl.ds(idx_ref[0, 0, n * DEG + j], 1), :]
                for j in range(DEG)]
        while len(vals) > 1:
            vals = [vals[i] + vals[i + 1] for i in range(0, len(vals), 2)]
        o_ref[pl.ds(n, 1), :] = vals[0] * (1.0 / DEG)
        return carry

    lax.fori_loop(0, TCB, node_body, 0, unroll=False)


def _gmean_tc(neighs_tail, x):
    return pl.pallas_call(
        _gmean_tc_body,
        grid=(NTC // TCB,),
        in_specs=[
            pl.BlockSpec((1, 1, TCB * DEG), lambda i: (i, 0, 0),
                         memory_space=pltpu.SMEM),
            pl.BlockSpec((N, D), lambda i: (0, 0)),
        ],
        out_specs=pl.BlockSpec((TCB, D), lambda i: (i, 0)),
        out_shape=jax.ShapeDtypeStruct((NTC, D), jnp.float32),
    )(neighs_tail.reshape(NTC // TCB, 1, TCB * DEG), x)


@jax.jit
def kernel(x, neighs, W1, b1, W2, b2):
    neighs_sc = neighs[:NSC].reshape(-1)
    neighs_tc = neighs[NSC:]
    h1_sc = _gmean_sc(neighs_sc, x)
    h1_tc = _gmean_tc(neighs_tc, x)
    h1 = jnp.concatenate([h1_sc, h1_tc], axis=0)
    x1 = _linear(h1, W1, b1)
    h2_sc = _gmean_sc(neighs_sc, x1)
    h2_tc = _gmean_tc(neighs_tc, x1)
    h2 = jnp.concatenate([h2_sc, h2_tc], axis=0)
    x2 = _linear(h2, W2, b2)
    return (x1, x2)


# speedup vs baseline: 3.8196x; 1.0199x over previous
import functools

import jax
import jax.numpy as jnp
from jax import lax
from jax.experimental import pallas as pl
from jax.experimental.pallas import tpu as pltpu
from jax.experimental.pallas import tpu_sc as plsc

N = 10000
DEG = 16
D = 256
HALF = D // 2
LANES = 16
SEGS = HALF // LANES
NC = 2
NS = 16
CHUNK = 8
NBUF = 2
NSC = 7936
NPT = NSC // NS
NCHUNKS = NPT // CHUNK
NTC = N - NSC
RPT = 632
RPT_LAST = N - (NS - 1) * RPT

_mesh = plsc.VectorSubcoreMesh(core_axis_name="c", subcore_axis_name="s")


def _gmean_half(cbase, s, neighs_hbm, x_hbm, out_hbm, idx_all, shared,
                rows, acc, gsem, osem):
    @pl.when(s < NS - 1)
    def _():
        pltpu.sync_copy(
            x_hbm.at[pl.ds(s * RPT, RPT), pl.ds(cbase, HALF)],
            shared.at[pl.ds(s * RPT, RPT)])

    @pl.when(s == NS - 1)
    def _():
        pltpu.sync_copy(
            x_hbm.at[pl.ds((NS - 1) * RPT, RPT_LAST), pl.ds(cbase, HALF)],
            shared.at[pl.ds((NS - 1) * RPT, RPT_LAST)])

    plsc.subcore_barrier()

    base = s * NPT
    pltpu.sync_copy(neighs_hbm.at[pl.ds(base * DEG, NPT * DEG)], idx_all)

    def start_gather(k, b):
        pltpu.async_copy(
            shared.at[idx_all.at[pl.ds(k * CHUNK * DEG, CHUNK * DEG)]],
            rows[b], gsem[b])

    for kp in range(NBUF - 1):
        start_gather(kp, kp)

    def outer(k0, carry):
        for b in range(NBUF):
            k = k0 + b

            @pl.when(k + NBUF - 1 < NCHUNKS)
            def _():
                start_gather(k + NBUF - 1, (b + NBUF - 1) % NBUF)

            pltpu.make_async_copy(
                shared.at[idx_all.at[pl.ds(0, CHUNK * DEG)]], rows[b],
                gsem[b]).wait()

            @pl.when(k >= NBUF)
            def _():
                pltpu.make_async_copy(
                    acc[b],
                    out_hbm.at[pl.ds(base, CHUNK), pl.ds(cbase, HALF)],
                    osem[b]).wait()

            def col_body(v, carry2):
                c0 = v * LANES
                for c in range(CHUNK):
                    vals = [rows[b][c * DEG + j, pl.ds(c0, LANES)]
                            for j in range(DEG)]
                    while len(vals) > 1:
                        vals = [vals[i] + vals[i + 1]
                                for i in range(0, len(vals), 2)]
                    acc[b][c, pl.ds(c0, LANES)] = vals[0] * (1.0 / DEG)
                return carry2

            lax.fori_loop(0, SEGS, col_body, 0, unroll=False)
            pltpu.async_copy(
                acc[b],
                out_hbm.at[pl.ds(base + k * CHUNK, CHUNK),
                           pl.ds(cbase, HALF)],
                osem[b])
        return carry

    lax.fori_loop(0, NCHUNKS // NBUF, lambda i, c: outer(NBUF * i, c), 0,
                  unroll=False)
    for b in range(NBUF):
        pltpu.make_async_copy(
            acc[b], out_hbm.at[pl.ds(base, CHUNK), pl.ds(cbase, HALF)],
            osem[b]).wait()


@functools.partial(
    pl.kernel,
    out_type=jax.ShapeDtypeStruct((NSC, D), jnp.float32),
    mesh=_mesh,
    scratch_types=[
        pltpu.VMEM((NPT * DEG,), jnp.int32),
        pltpu.VMEM_SHARED((N, HALF), jnp.float32),
        *[pltpu.VMEM((CHUNK * DEG, HALF), jnp.float32)
          for _ in range(NBUF)],
        *[pltpu.VMEM((CHUNK, HALF), jnp.float32) for _ in range(NBUF)],
        *[pltpu.SemaphoreType.DMA for _ in range(2 * NBUF)],
    ],
)
def _gmean_sc(neighs_hbm, x_hbm, out_hbm, idx_all, shared, *bufs):
    rows = tuple(bufs[0:NBUF])
    acc = tuple(bufs[NBUF:2 * NBUF])
    gsem = tuple(bufs[2 * NBUF:3 * NBUF])
    osem = tuple(bufs[3 * NBUF:4 * NBUF])
    c = lax.axis_index("c")
    s = lax.axis_index("s")

    @pl.when(c == 0)
    def _():
        _gmean_half(0, s, neighs_hbm, x_hbm, out_hbm, idx_all, shared,
                    rows, acc, gsem, osem)

    @pl.when(c == 1)
    def _():
        _gmean_half(HALF, s, neighs_hbm, x_hbm, out_hbm, idx_all, shared,
                    rows, acc, gsem, osem)


BN = 2000


def _linear_body(h_ref, w_ref, b_ref, o_ref):
    o_ref[...] = (
        lax.dot_general(
            h_ref[...].astype(jnp.bfloat16),
            w_ref[...].astype(jnp.bfloat16),
            (((1,), (1,)), ((), ())),
            preferred_element_type=jnp.float32,
        )
        + b_ref[...]
    )


def _linear(h, W, b):
    return pl.pallas_call(
        _linear_body,
        grid=(N // BN,),
        in_specs=[
            pl.BlockSpec((BN, D), lambda i: (i, 0)),
            pl.BlockSpec((D, D), lambda i: (0, 0)),
            pl.BlockSpec((1, D), lambda i: (0, 0)),
        ],
        out_specs=pl.BlockSpec((BN, D), lambda i: (i, 0)),
        out_shape=jax.ShapeDtypeStruct((N, D), jnp.float32),
    )(h, W, b[None, :])


TCB = 344


def _gmean_tc_body(idx_ref, x_ref, o_ref):
    def node_body(n, carry):
        vals = [x_ref[pl.ds(idx_ref[0, 0, n * DEG + j], 1), :]
                for j in range(DEG)]
        while len(vals) > 1:
            vals = [vals[i] + vals[i + 1] for i in range(0, len(vals), 2)]
        o_ref[pl.ds(n, 1), :] = vals[0] * (1.0 / DEG)
        return carry

    lax.fori_loop(0, TCB, node_body, 0, unroll=False)


def _gmean_tc(neighs_tail, x):
    return pl.pallas_call(
        _gmean_tc_body,
        grid=(NTC // TCB,),
        in_specs=[
            pl.BlockSpec((1, 1, TCB * DEG), lambda i: (i, 0, 0),
                         memory_space=pltpu.SMEM),
            pl.BlockSpec((N, D), lambda i: (0, 0)),
        ],
        out_specs=pl.BlockSpec((TCB, D), lambda i: (i, 0)),
        out_shape=jax.ShapeDtypeStruct((NTC, D), jnp.float32),
    )(neighs_tail.reshape(NTC // TCB, 1, TCB * DEG), x)


@jax.jit
def kernel(x, neighs, W1, b1, W2, b2):
    neighs_sc = neighs[:NSC].reshape(-1)
    neighs_tc = neighs[NSC:]
    h1_sc = _gmean_sc(neighs_sc, x)
    h1_tc = _gmean_tc(neighs_tc, x)
    h1 = jnp.concatenate([h1_sc, h1_tc], axis=0)
    x1 = _linear(h1, W1, b1)
    h2_sc = _gmean_sc(neighs_sc, x1)
    h2_tc = _gmean_tc(neighs_tc, x1)
    h2 = jnp.concatenate([h2_sc, h2_tc], axis=0)
    x2 = _linear(h2, W2, b2)
    return (x1, x2)


# speedup vs baseline: 3.9256x; 1.0278x over previous
import functools

import jax
import jax.numpy as jnp
from jax import lax
from jax.experimental import pallas as pl
from jax.experimental.pallas import tpu as pltpu
from jax.experimental.pallas import tpu_sc as plsc

N = 10000
DEG = 16
D = 256
HALF = D // 2
LANES = 16
SEGS = HALF // LANES
NC = 2
NS = 16
CHUNK = 8
NBUF = 2
NSC = 7936
NPT = NSC // NS
NCHUNKS = NPT // CHUNK
NTC = N - NSC
RPT = 632
RPT_LAST = N - (NS - 1) * RPT

_mesh = plsc.VectorSubcoreMesh(core_axis_name="c", subcore_axis_name="s")


def _gmean_half(cbase, s, neighs_hbm, x_hbm, out_hbm, idx_all, shared,
                rows, acc, gsem, osem):
    @pl.when(s < NS - 1)
    def _():
        pltpu.sync_copy(
            x_hbm.at[pl.ds(s * RPT, RPT), pl.ds(cbase, HALF)],
            shared.at[pl.ds(s * RPT, RPT)])

    @pl.when(s == NS - 1)
    def _():
        pltpu.sync_copy(
            x_hbm.at[pl.ds((NS - 1) * RPT, RPT_LAST), pl.ds(cbase, HALF)],
            shared.at[pl.ds((NS - 1) * RPT, RPT_LAST)])

    plsc.subcore_barrier()

    base = s * NPT
    pltpu.sync_copy(neighs_hbm.at[pl.ds(base * DEG, NPT * DEG)], idx_all)

    def start_gather(k, b):
        pltpu.async_copy(
            shared.at[idx_all.at[pl.ds(k * CHUNK * DEG, CHUNK * DEG)]],
            rows[b], gsem[b])

    for kp in range(NBUF - 1):
        start_gather(kp, kp)

    def outer(k0, carry):
        for b in range(NBUF):
            k = k0 + b

            @pl.when(k + NBUF - 1 < NCHUNKS)
            def _():
                start_gather(k + NBUF - 1, (b + NBUF - 1) % NBUF)

            pltpu.make_async_copy(
                shared.at[idx_all.at[pl.ds(0, CHUNK * DEG)]], rows[b],
                gsem[b]).wait()

            @pl.when(k >= NBUF)
            def _():
                pltpu.make_async_copy(
                    acc[b],
                    out_hbm.at[pl.ds(base, CHUNK), pl.ds(cbase, HALF)],
                    osem[b]).wait()

            def col_body(v, carry2):
                c0 = v * LANES
                for c in range(CHUNK):
                    vals = [rows[b][c * DEG + j, pl.ds(c0, LANES)]
                            for j in range(DEG)]
                    while len(vals) > 1:
                        vals = [vals[i] + vals[i + 1]
                                for i in range(0, len(vals), 2)]
                    acc[b][c, pl.ds(c0, LANES)] = vals[0] * (1.0 / DEG)
                return carry2

            lax.fori_loop(0, SEGS, col_body, 0, unroll=False)
            pltpu.async_copy(
                acc[b],
                out_hbm.at[pl.ds(base + k * CHUNK, CHUNK),
                           pl.ds(cbase, HALF)],
                osem[b])
        return carry

    lax.fori_loop(0, NCHUNKS // NBUF, lambda i, c: outer(NBUF * i, c), 0,
                  unroll=False)
    for b in range(NBUF):
        pltpu.make_async_copy(
            acc[b], out_hbm.at[pl.ds(base, CHUNK), pl.ds(cbase, HALF)],
            osem[b]).wait()


@functools.partial(
    pl.kernel,
    out_type=jax.ShapeDtypeStruct((NSC, D), jnp.float32),
    mesh=_mesh,
    scratch_types=[
        pltpu.VMEM((NPT * DEG,), jnp.int32),
        pltpu.VMEM_SHARED((N, HALF), jnp.float32),
        *[pltpu.VMEM((CHUNK * DEG, HALF), jnp.float32)
          for _ in range(NBUF)],
        *[pltpu.VMEM((CHUNK, HALF), jnp.float32) for _ in range(NBUF)],
        *[pltpu.SemaphoreType.DMA for _ in range(2 * NBUF)],
    ],
)
def _gmean_sc(neighs_hbm, x_hbm, out_hbm, idx_all, shared, *bufs):
    rows = tuple(bufs[0:NBUF])
    acc = tuple(bufs[NBUF:2 * NBUF])
    gsem = tuple(bufs[2 * NBUF:3 * NBUF])
    osem = tuple(bufs[3 * NBUF:4 * NBUF])
    c = lax.axis_index("c")
    s = lax.axis_index("s")

    @pl.when(c == 0)
    def _():
        _gmean_half(0, s, neighs_hbm, x_hbm, out_hbm, idx_all, shared,
                    rows, acc, gsem, osem)

    @pl.when(c == 1)
    def _():
        _gmean_half(HALF, s, neighs_hbm, x_hbm, out_hbm, idx_all, shared,
                    rows, acc, gsem, osem)


BN = 5000


def _linear_body(h_ref, w_ref, b_ref, o_ref):
    o_ref[...] = (
        lax.dot_general(
            h_ref[...].astype(jnp.bfloat16),
            w_ref[...].astype(jnp.bfloat16),
            (((1,), (1,)), ((), ())),
            preferred_element_type=jnp.float32,
        )
        + b_ref[...]
    )


def _linear(h, W, b):
    return pl.pallas_call(
        _linear_body,
        grid=(N // BN,),
        in_specs=[
            pl.BlockSpec((BN, D), lambda i: (i, 0)),
            pl.BlockSpec((D, D), lambda i: (0, 0)),
            pl.BlockSpec((1, D), lambda i: (0, 0)),
        ],
        out_specs=pl.BlockSpec((BN, D), lambda i: (i, 0)),
        out_shape=jax.ShapeDtypeStruct((N, D), jnp.float32),
    )(h, W, b[None, :])


TCB = 344


def _gmean_tc_body(idx_ref, x_ref, o_ref):
    def node_body(n, carry):
        vals = [x_ref[pl.ds(idx_ref[0, 0, n * DEG + j], 1), :]
                for j in range(DEG)]
        while len(vals) > 1:
            vals = [vals[i] + vals[i + 1] for i in range(0, len(vals), 2)]
        o_ref[pl.ds(n, 1), :] = vals[0] * (1.0 / DEG)
        return carry

    lax.fori_loop(0, TCB, node_body, 0, unroll=False)


def _gmean_tc(neighs_tail, x):
    return pl.pallas_call(
        _gmean_tc_body,
        grid=(NTC // TCB,),
        in_specs=[
            pl.BlockSpec((1, 1, TCB * DEG), lambda i: (i, 0, 0),
                         memory_space=pltpu.SMEM),
            pl.BlockSpec((N, D), lambda i: (0, 0)),
        ],
        out_specs=pl.BlockSpec((TCB, D), lambda i: (i, 0)),
        out_shape=jax.ShapeDtypeStruct((NTC, D), jnp.float32),
    )(neighs_tail.reshape(NTC // TCB, 1, TCB * DEG), x)


@jax.jit
def kernel(x, neighs, W1, b1, W2, b2):
    neighs_sc = neighs[:NSC].reshape(-1)
    neighs_tc = neighs[NSC:]
    h1_sc = _gmean_sc(neighs_sc, x)
    h1_tc = _gmean_tc(neighs_tc, x)
    h1 = jnp.concatenate([h1_sc, h1_tc], axis=0)
    x1 = _linear(h1, W1, b1)
    h2_sc = _gmean_sc(neighs_sc, x1)
    h2_tc = _gmean_tc(neighs_tc, x1)
    h2 = jnp.concatenate([h2_sc, h2_tc], axis=0)
    x2 = _linear(h2, W2, b2)
    return (x1, x2)
